# baseline (speedup 1.0000x reference)
import functools

import jax
import jax.numpy as jnp
from jax import lax
from jax.experimental import pallas as pl
from jax.experimental.pallas import tpu as pltpu
from jax.experimental.pallas import tpu_sc as plsc

B = 16384
D = 64
L = 16
NC = 2
NS = 16
NW = NC * NS
BPW = B // NW
CHUNK = 128
NCHUNK = BPW // CHUNK
GROUPS = BPW // L


def _make_sc_kernel():
    mesh = plsc.VectorSubcoreMesh(core_axis_name="c", subcore_axis_name="s")

    @functools.partial(
        pl.kernel,
        mesh=mesh,
        out_type=jax.ShapeDtypeStruct((B,), jnp.float32),
        compiler_params=pltpu.CompilerParams(needs_layout_passes=False,
                                              use_tc_tiling_on_sc=False),
        scratch_types=[
            pltpu.VMEM((NCHUNK, CHUNK), jnp.int32),
            pltpu.VMEM((NCHUNK, CHUNK), jnp.int32),
            pltpu.VMEM((BPW, D), jnp.float32),
            pltpu.VMEM((BPW, D), jnp.float32),
            pltpu.VMEM((BPW,), jnp.float32),
            pltpu.SemaphoreType.DMA,
        ],
    )
    def sc_body(ut_hbm, it_hbm, uix_hbm, iix_hbm, out_hbm,
                uix_v, iix_v, ur_v, ir_v, out_v, sem):
        wid = lax.axis_index("s") * NC + lax.axis_index("c")
        pltpu.sync_copy(uix_hbm.at[wid], uix_v)
        pltpu.sync_copy(iix_hbm.at[wid], iix_v)
        copies = []
        for j in range(NCHUNK):
            copies.append(pltpu.async_copy(
                ut_hbm.at[uix_v.at[j]], ur_v.at[pl.ds(j * CHUNK, CHUNK)], sem))
            copies.append(pltpu.async_copy(
                it_hbm.at[iix_v.at[j]], ir_v.at[pl.ds(j * CHUNK, CHUNK)], sem))
        for c in copies:
            c.wait()

        def group(g, carry):
            row0 = pl.multiple_of(g * L, L)
            rows = row0 + lax.iota(jnp.int32, L)
            acc = jnp.zeros((L,), jnp.float32)
            for col in range(D):
                cc = jnp.full((L,), col, jnp.int32)
                acc = acc + (plsc.load_gather(ur_v, [rows, cc])
                             * plsc.load_gather(ir_v, [rows, cc]))
            out_v[pl.ds(row0, L)] = acc
            return carry

        lax.fori_loop(0, GROUPS, group, 0)
        base = pl.multiple_of(wid * BPW, BPW)
        pltpu.sync_copy(out_v, out_hbm.at[pl.ds(base, BPW)])

    return sc_body


_sc_kernel = _make_sc_kernel()


def kernel(inputs, user_embedding, item_embedding):
    uix = inputs[:, 0].reshape(NW, NCHUNK, CHUNK)
    iix = inputs[:, 1].reshape(NW, NCHUNK, CHUNK)
    return _sc_kernel(user_embedding, item_embedding, uix, iix)
